# Initial kernel scaffold; baseline (speedup 1.0000x reference)
import jax
import jax.numpy as jnp
from jax.experimental import pallas as pl


def kernel(x, edge_list, W1, b1, W2, b2, W3, b3):
    raise NotImplementedError("write your pallas kernel here")


# speedup vs baseline: 13.3183x; 13.3183x over previous
import functools

import jax
import jax.numpy as jnp
from jax import lax
from jax.experimental import pallas as pl
from jax.experimental.pallas import tpu as pltpu
from jax.experimental.pallas import tpu_sc as plsc

NC = 2
NS = 16
NW = NC * NS
LANES = 16


def _sc_agg(src, dst, table=None, hist_shape=None):
    gather = table is not None
    if gather:
        N, D = table.shape
    else:
        N, D = hist_shape
    E = dst.shape[0]
    assert E % NW == 0
    EW = E // NW
    C = 80
    assert EW % C == 0 and C % 8 == 0
    nch = EW // C
    assert N % NS == 0
    TR = N // NS

    mesh = plsc.VectorSubcoreMesh(core_axis_name="c", subcore_axis_name="s")

    scratch = [
        pltpu.VMEM((C,), jnp.int32),
        pltpu.VMEM((C,), jnp.int32),
        pltpu.VMEM((C, D), jnp.float32),
        pltpu.VMEM_SHARED((N, D), jnp.float32),
        pltpu.SemaphoreType.DMA,
    ]

    def body(table_h, src_h, dst_h, out_h, sidx, didx, rows, acc, sem):
        c = lax.axis_index("c")
        s = lax.axis_index("s")
        w = c * NS + s

        nv = D // LANES

        def fill_rows(val):
            def fb(t, carry):
                r = t // nv
                col = (t % nv) * LANES
                rows[r, pl.ds(col, LANES)] = jnp.full((LANES,), val, jnp.float32)
                return carry
            lax.fori_loop(0, C * nv, fb, 0)

        fill_rows(0.0)
        r0 = s * TR
        off = 0
        while off < TR:
            m = min(C, TR - off)
            pltpu.sync_copy(rows.at[pl.ds(0, m)], acc.at[pl.ds(r0 + off, m)])
            off += m
        if not gather:
            fill_rows(1.0)
        plsc.subcore_barrier()

        base0 = w * EW

        def chunk(i, carry):
            base = base0 + i * C
            pltpu.sync_copy(dst_h.at[pl.ds(base, C)], didx)
            if gather:
                pltpu.sync_copy(src_h.at[pl.ds(base, C)], sidx)
                pltpu.async_copy(table_h.at[sidx], rows, sem).wait()
            pltpu.sync_copy(rows, acc.at[didx], add=True)
            return carry

        lax.fori_loop(0, nch, chunk, 0)
        plsc.subcore_barrier()

        pltpu.sync_copy(acc.at[pl.ds(r0, TR)], out_h.at[c, s])

    if gather:
        args = (table, src, dst)

        def k_gather(table_h, src_h, dst_h, out_h, sidx, didx, rows, acc, sem):
            body(table_h, src_h, dst_h, out_h, sidx, didx, rows, acc, sem)

        fn = k_gather
    else:
        args = (dst,)

        def k_hist(dst_h, out_h, sidx, didx, rows, acc, sem):
            body(None, None, dst_h, out_h, sidx, didx, rows, acc, sem)

        fn = k_hist

    run = functools.partial(
        pl.kernel,
        mesh=mesh,
        out_type=jax.ShapeDtypeStruct((NC, NS, TR, D), jnp.float32),
        scratch_types=scratch,
        compiler_params=pltpu.CompilerParams(use_tc_tiling_on_sc=False),
    )(fn)
    return run(*args).reshape(NC, N, D)


_ROWS = 1000


def _dis_of(degp_ref):
    deg = degp_ref[0, :, 0] + degp_ref[1, :, 0] + 1.0
    return lax.rsqrt(deg)[:, None]


def _prep_body(degp_ref, x_ref, w_ref, o_ref):
    dis = _dis_of(degp_ref)
    o_ref[...] = jnp.dot(x_ref[...], w_ref[...],
                         preferred_element_type=jnp.float32) * dis


def _tc_prep(degp, x, W):
    N, Din = x.shape
    Dh = W.shape[1]
    return pl.pallas_call(
        _prep_body,
        grid=(N // _ROWS,),
        in_specs=[
            pl.BlockSpec((NC, _ROWS, LANES), lambda i: (0, i, 0)),
            pl.BlockSpec((_ROWS, Din), lambda i: (i, 0)),
            pl.BlockSpec((Din, Dh), lambda i: (0, 0)),
        ],
        out_specs=pl.BlockSpec((_ROWS, Dh), lambda i: (i, 0)),
        out_shape=jax.ShapeDtypeStruct((N, Dh), jnp.float32),
    )(degp, x, W)


def _fp_body(aggp_ref, hp_ref, degp_ref, b_ref, w_ref, o_ref):
    dis = _dis_of(degp_ref)
    a = aggp_ref[0] + aggp_ref[1] + hp_ref[...]
    y = jnp.maximum(dis * a + b_ref[...], 0.0)
    o_ref[...] = jnp.dot(y, w_ref[...],
                         preferred_element_type=jnp.float32) * dis


def _tc_finish_prep(aggp, hp, degp, b, W):
    N, D = hp.shape
    K = W.shape[1]
    return pl.pallas_call(
        _fp_body,
        grid=(N // _ROWS,),
        in_specs=[
            pl.BlockSpec((NC, _ROWS, D), lambda i: (0, i, 0)),
            pl.BlockSpec((_ROWS, D), lambda i: (i, 0)),
            pl.BlockSpec((NC, _ROWS, LANES), lambda i: (0, i, 0)),
            pl.BlockSpec((1, D), lambda i: (0, 0)),
            pl.BlockSpec((D, K), lambda i: (0, 0)),
        ],
        out_specs=pl.BlockSpec((_ROWS, K), lambda i: (i, 0)),
        out_shape=jax.ShapeDtypeStruct((N, K), jnp.float32),
    )(aggp, hp, degp, b.reshape(1, D), W)


def _final_body(aggp_ref, hp_ref, degp_ref, b_ref, o_ref):
    dis = _dis_of(degp_ref)
    a = aggp_ref[0] + aggp_ref[1] + hp_ref[...]
    y = dis * a + b_ref[0, 0]
    o_ref[...] = y[:, :1]


def _tc_final(aggp, hp, degp, b):
    N = hp.shape[0]
    return pl.pallas_call(
        _final_body,
        grid=(N // _ROWS,),
        in_specs=[
            pl.BlockSpec((NC, _ROWS, LANES), lambda i: (0, i, 0)),
            pl.BlockSpec((_ROWS, LANES), lambda i: (i, 0)),
            pl.BlockSpec((NC, _ROWS, LANES), lambda i: (0, i, 0)),
            pl.BlockSpec((1, 1), lambda i: (0, 0)),
        ],
        out_specs=pl.BlockSpec((_ROWS, 1), lambda i: (i, 0)),
        out_shape=jax.ShapeDtypeStruct((N, 1), jnp.float32),
    )(aggp, hp, degp, b.reshape(1, 1))


def kernel(x, edge_list, W1, b1, W2, b2, W3, b3):
    N = x.shape[0]
    src = edge_list[0]
    dst = edge_list[1]
    W3p = jnp.pad(W3, ((0, 0), (0, LANES - W3.shape[1])))
    b3p = jnp.pad(b3, (0, 0))

    degp = _sc_agg(src, dst, hist_shape=(N, LANES))
    h1p = _tc_prep(degp, x, W1)
    agg1 = _sc_agg(src, dst, table=h1p)
    h2p = _tc_finish_prep(agg1, h1p, degp, b1, W2)
    agg2 = _sc_agg(src, dst, table=h2p)
    h3p = _tc_finish_prep(agg2, h2p, degp, b2, W3p)
    agg3 = _sc_agg(src, dst, table=h3p)
    return _tc_final(agg3, h3p, degp, b3p)


# speedup vs baseline: 33.9609x; 2.5499x over previous
import functools

import jax
import jax.numpy as jnp
from jax import lax
from jax.experimental import pallas as pl
from jax.experimental.pallas import tpu as pltpu
from jax.experimental.pallas import tpu_sc as plsc

NC = 2
NS = 16
NW = NC * NS
LANES = 16


def _sc_agg(src, dst, table=None, hist_shape=None):
    gather = table is not None
    if gather:
        N, D = table.shape
    else:
        N, D = hist_shape
    E = dst.shape[0]
    assert E % NW == 0
    EW = E // NW
    C = 80
    NB = 5
    assert EW % C == 0 and C % 8 == 0
    nch = EW // C
    assert nch % NB == 0
    ng = nch // NB
    assert N % NS == 0
    TR = N // NS

    mesh = plsc.VectorSubcoreMesh(core_axis_name="c", subcore_axis_name="s")

    src3 = src.reshape(NW, nch, C) if gather else None
    dst3 = dst.reshape(NW, nch, C)

    scratch = [
        pltpu.VMEM((nch, C), jnp.int32),
        pltpu.VMEM((nch, C), jnp.int32),
        pltpu.VMEM((2, NB, C, D), jnp.float32),
        pltpu.VMEM_SHARED((N, D), jnp.float32),
        pltpu.SemaphoreType.DMA,
        pltpu.SemaphoreType.DMA,
    ]

    def body(table_h, src3_h, dst3_h, out_h, sidx, didx, bufs, acc, gsem, ssem):
        c = lax.axis_index("c")
        s = lax.axis_index("s")
        w = c * NS + s

        nv = D // LANES

        def fill_buf0(val):
            def fb(t, carry):
                r = t // nv
                col = (t % nv) * LANES
                bufs[0, 0, r, pl.ds(col, LANES)] = jnp.full(
                    (LANES,), val, jnp.float32)
                return carry
            lax.fori_loop(0, C * nv, fb, 0)

        fill_buf0(0.0)
        r0 = s * TR
        off = 0
        while off < TR:
            m = min(C, TR - off)
            pltpu.sync_copy(bufs.at[0, 0, pl.ds(0, m)],
                            acc.at[pl.ds(r0 + off, m)])
            off += m

        pltpu.sync_copy(dst3_h.at[w], didx)
        if gather:
            pltpu.sync_copy(src3_h.at[w], sidx)
        else:
            fill_buf0(1.0)
        plsc.subcore_barrier()

        if gather:
            def start_gathers(g, p):
                for j in range(NB):
                    pltpu.async_copy(table_h.at[sidx.at[g * NB + j]],
                                     bufs.at[p, j], gsem)

            def drain(sem, p, dummy_dst):
                for j in range(NB):
                    pltpu.make_async_copy(
                        table_h.at[pl.ds(0, C)], dummy_dst.at[p, j], sem
                    ).wait()

            start_gathers(0, 0)

            def group(t, carry):
                p = lax.rem(t, 2)
                q = 1 - p
                drain(gsem, p, bufs)
                for j in range(NB):
                    pltpu.async_copy(bufs.at[p, j],
                                     acc.at[didx.at[t * NB + j]],
                                     ssem, add=True)

                @pl.when(t + 1 < ng)
                def _():
                    start_gathers(t + 1, q)

                drain(ssem, p, bufs)
                return carry

            lax.fori_loop(0, ng, group, 0)
        else:
            def group(t, carry):
                for j in range(NB):
                    pltpu.async_copy(bufs.at[0, 0],
                                     acc.at[didx.at[t * NB + j]],
                                     ssem, add=True)
                for j in range(NB):
                    pltpu.make_async_copy(
                        bufs.at[0, 0], acc.at[pl.ds(0, C)], ssem
                    ).wait()
                return carry

            lax.fori_loop(0, ng, group, 0)

        plsc.subcore_barrier()
        pltpu.sync_copy(acc.at[pl.ds(r0, TR)], out_h.at[c, s])

    if gather:
        args = (table, src3, dst3)

        def k_gather(table_h, src3_h, dst3_h, out_h,
                     sidx, didx, bufs, acc, gsem, ssem):
            body(table_h, src3_h, dst3_h, out_h,
                 sidx, didx, bufs, acc, gsem, ssem)

        fn = k_gather
    else:
        args = (dst3,)

        def k_hist(dst3_h, out_h, sidx, didx, bufs, acc, gsem, ssem):
            body(None, None, dst3_h, out_h,
                 sidx, didx, bufs, acc, gsem, ssem)

        fn = k_hist

    run = functools.partial(
        pl.kernel,
        mesh=mesh,
        out_type=jax.ShapeDtypeStruct((NC, NS, TR, D), jnp.float32),
        scratch_types=scratch,
        compiler_params=pltpu.CompilerParams(use_tc_tiling_on_sc=False),
    )(fn)
    return run(*args).reshape(NC, N, D)


_ROWS = 1000


def _dis_of(degp_ref):
    deg = degp_ref[0, :, 0] + degp_ref[1, :, 0] + 1.0
    return lax.rsqrt(deg)[:, None]


def _prep_body(degp_ref, x_ref, w_ref, o1_ref, o2_ref):
    dis = _dis_of(degp_ref)
    h = jnp.dot(x_ref[...], w_ref[...],
                preferred_element_type=jnp.float32) * dis
    half = h.shape[1] // 2
    o1_ref[...] = h[:, :half]
    o2_ref[...] = h[:, half:]


def _tc_prep(degp, x, W):
    N, Din = x.shape
    Dh = W.shape[1]
    half = Dh // 2
    return pl.pallas_call(
        _prep_body,
        grid=(N // _ROWS,),
        in_specs=[
            pl.BlockSpec((NC, _ROWS, LANES), lambda i: (0, i, 0)),
            pl.BlockSpec((_ROWS, Din), lambda i: (i, 0)),
            pl.BlockSpec((Din, Dh), lambda i: (0, 0)),
        ],
        out_specs=[
            pl.BlockSpec((_ROWS, half), lambda i: (i, 0)),
            pl.BlockSpec((_ROWS, half), lambda i: (i, 0)),
        ],
        out_shape=[
            jax.ShapeDtypeStruct((N, half), jnp.float32),
            jax.ShapeDtypeStruct((N, half), jnp.float32),
        ],
    )(degp, x, W)


def _fp1_body(aggA_ref, aggB_ref, hpA_ref, hpB_ref, degp_ref, b_ref, w_ref,
              o_ref):
    dis = _dis_of(degp_ref)
    half = hpA_ref.shape[1]
    aA = aggA_ref[0] + aggA_ref[1] + hpA_ref[...]
    aB = aggB_ref[0] + aggB_ref[1] + hpB_ref[...]
    yA = jnp.maximum(dis * aA + b_ref[:, :half], 0.0)
    yB = jnp.maximum(dis * aB + b_ref[:, half:], 0.0)
    y = jnp.concatenate([yA, yB], axis=1)
    o_ref[...] = jnp.dot(y, w_ref[...],
                         preferred_element_type=jnp.float32) * dis


def _tc_finish_prep1(aggA, aggB, hpA, hpB, degp, b, W):
    N, half = hpA.shape
    D = 2 * half
    K = W.shape[1]
    return pl.pallas_call(
        _fp1_body,
        grid=(N // _ROWS,),
        in_specs=[
            pl.BlockSpec((NC, _ROWS, half), lambda i: (0, i, 0)),
            pl.BlockSpec((NC, _ROWS, half), lambda i: (0, i, 0)),
            pl.BlockSpec((_ROWS, half), lambda i: (i, 0)),
            pl.BlockSpec((_ROWS, half), lambda i: (i, 0)),
            pl.BlockSpec((NC, _ROWS, LANES), lambda i: (0, i, 0)),
            pl.BlockSpec((1, D), lambda i: (0, 0)),
            pl.BlockSpec((D, K), lambda i: (0, 0)),
        ],
        out_specs=pl.BlockSpec((_ROWS, K), lambda i: (i, 0)),
        out_shape=jax.ShapeDtypeStruct((N, K), jnp.float32),
    )(aggA, aggB, hpA, hpB, degp, b.reshape(1, D), W)


def _fp_body(aggp_ref, hp_ref, degp_ref, b_ref, w_ref, o_ref):
    dis = _dis_of(degp_ref)
    a = aggp_ref[0] + aggp_ref[1] + hp_ref[...]
    y = jnp.maximum(dis * a + b_ref[...], 0.0)
    o_ref[...] = jnp.dot(y, w_ref[...],
                         preferred_element_type=jnp.float32) * dis


def _tc_finish_prep(aggp, hp, degp, b, W):
    N, D = hp.shape
    K = W.shape[1]
    return pl.pallas_call(
        _fp_body,
        grid=(N // _ROWS,),
        in_specs=[
            pl.BlockSpec((NC, _ROWS, D), lambda i: (0, i, 0)),
            pl.BlockSpec((_ROWS, D), lambda i: (i, 0)),
            pl.BlockSpec((NC, _ROWS, LANES), lambda i: (0, i, 0)),
            pl.BlockSpec((1, D), lambda i: (0, 0)),
            pl.BlockSpec((D, K), lambda i: (0, 0)),
        ],
        out_specs=pl.BlockSpec((_ROWS, K), lambda i: (i, 0)),
        out_shape=jax.ShapeDtypeStruct((N, K), jnp.float32),
    )(aggp, hp, degp, b.reshape(1, D), W)


def _final_body(aggp_ref, hp_ref, degp_ref, b_ref, o_ref):
    dis = _dis_of(degp_ref)
    a = aggp_ref[0] + aggp_ref[1] + hp_ref[...]
    y = dis * a + b_ref[0, 0]
    o_ref[...] = y[:, :1]


def _tc_final(aggp, hp, degp, b):
    N = hp.shape[0]
    return pl.pallas_call(
        _final_body,
        grid=(N // _ROWS,),
        in_specs=[
            pl.BlockSpec((NC, _ROWS, LANES), lambda i: (0, i, 0)),
            pl.BlockSpec((_ROWS, LANES), lambda i: (i, 0)),
            pl.BlockSpec((NC, _ROWS, LANES), lambda i: (0, i, 0)),
            pl.BlockSpec((1, 1), lambda i: (0, 0)),
        ],
        out_specs=pl.BlockSpec((_ROWS, 1), lambda i: (i, 0)),
        out_shape=jax.ShapeDtypeStruct((N, 1), jnp.float32),
    )(aggp, hp, degp, b.reshape(1, 1))


def kernel(x, edge_list, W1, b1, W2, b2, W3, b3):
    N = x.shape[0]
    src = edge_list[0]
    dst = edge_list[1]
    W3p = jnp.pad(W3, ((0, 0), (0, LANES - W3.shape[1])))
    b3p = jnp.pad(b3, (0, 0))

    degp = _sc_agg(src, dst, hist_shape=(N, LANES))
    h1pA, h1pB = _tc_prep(degp, x, W1)
    agg1A = _sc_agg(src, dst, table=h1pA)
    agg1B = _sc_agg(src, dst, table=h1pB)
    h2p = _tc_finish_prep1(agg1A, agg1B, h1pA, h1pB, degp, b1, W2)
    agg2 = _sc_agg(src, dst, table=h2p)
    h3p = _tc_finish_prep(agg2, h2p, degp, b2, W3p)
    agg3 = _sc_agg(src, dst, table=h3p)
    return _tc_final(agg3, h3p, degp, b3p)
